# Initial kernel scaffold; baseline (speedup 1.0000x reference)
import jax
import jax.numpy as jnp
from jax.experimental import pallas as pl


def kernel(x, phi):
    raise NotImplementedError("write your pallas kernel here")


# speedup vs baseline: 3.6510x; 3.6510x over previous
import functools

import jax
import jax.numpy as jnp
from jax import lax
from jax.experimental import pallas as pl
from jax.experimental.pallas import tpu as pltpu
from jax.experimental.pallas import tpu_sc as plsc

B = 2
C = 192
H = 224
W = 224
HW = H * W

NUM_CORES = 2
NUM_SUBCORES = 16
LANES = 16

CPT = C // NUM_SUBCORES
PASSES = CPT // 2
CHUNK = 896
NCHUNK = HW // CHUNK
GROUPS = CHUNK // LANES
RCHUNK = 128
NRCHUNK = HW // RCHUNK


def _prep_pallas(phi):
    def body(phi_ref, o_ref):
        ph0 = phi_ref[0]
        ph1 = phi_ref[1]
        ii = lax.broadcasted_iota(jnp.int32, (H, W), 0).astype(jnp.float32)
        jj = lax.broadcasted_iota(jnp.int32, (H, W), 1).astype(jnp.float32)
        gi = ii + ph0
        gj = jj + ph1
        i0f = jnp.floor(gi)
        j0f = jnp.floor(gj)
        i0 = jnp.mod(i0f.astype(jnp.int32), H)
        j0 = jnp.mod(j0f.astype(jnp.int32), W)
        o_ref[0] = (i0 * W).astype(jnp.float32)
        o_ref[1] = j0.astype(jnp.float32)
        o_ref[2] = gi - i0f
        o_ref[3] = gj - j0f

    return pl.pallas_call(
        body,
        grid=(B,),
        in_specs=[pl.BlockSpec((None, 2, H, W), lambda b: (b, 0, 0, 0))],
        out_specs=pl.BlockSpec((None, 4, H, W), lambda b: (b, 0, 0, 0)),
        out_shape=jax.ShapeDtypeStruct((B, 4, H, W), jnp.float32),
    )(phi)


def _decode_group(iw_ref, s, g):
    base = g * LANES
    i0w = iw_ref[s, 0, pl.ds(base, LANES)].astype(jnp.int32)
    j0 = iw_ref[s, 1, pl.ds(base, LANES)].astype(jnp.int32)
    wi = iw_ref[s, 2, pl.ds(base, LANES)]
    wj = iw_ref[s, 3, pl.ds(base, LANES)]
    j1 = j0 + 1
    j1 = jnp.where(j1 == W, 0, j1)
    i1w = i0w + W
    i1w = jnp.where(i1w == HW, 0, i1w)
    u = 1.0 - wi
    v = 1.0 - wj
    return (
        (i0w + j0, u * v),
        (i0w + j1, u * wj),
        (i1w + j0, wi * v),
        (i1w + j1, wi * wj),
    )


def _sc_push(xr, iw):
    mesh = plsc.VectorSubcoreMesh(core_axis_name="c", subcore_axis_name="s")

    def body(x_hbm, iw_hbm, out_hbm, acc_a, acc_b, iwbuf, xbuf, rbuf, rsbuf,
             obuf, sh_r, sem_iw0, sem_iw1, sem_x0, sem_x1, sem_o0,
             sem_o1):
        cid = lax.axis_index("c")
        sid = lax.axis_index("s")
        sem_iw = (sem_iw0, sem_iw1)
        sem_x = (sem_x0, sem_x1)
        sem_o = (sem_o0, sem_o1)

        def iw_src(t):
            off = pl.multiple_of(t * CHUNK, 8)
            return iw_hbm.at[cid, :, pl.ds(off, CHUNK)]

        def start_iw(t, s):
            pltpu.async_copy(iw_src(t), iwbuf.at[s], sem_iw[s])

        def wait_iw(t, s):
            pltpu.make_async_copy(iw_src(t), iwbuf.at[s], sem_iw[s]).wait()

        def chunk_loop(compute_chunk, extra_start, extra_wait):
            start_iw(0, 0)
            extra_start(0, 0)
            start_iw(1, 1)
            extra_start(1, 1)

            def pair_body(tp, carry):
                for s in (0, 1):
                    t = tp * 2 + s
                    wait_iw(t, s)
                    extra_wait(t, s)
                    compute_chunk(t, s)

                    @pl.when(t + 2 < NCHUNK)
                    def _():
                        start_iw(t + 2, s)
                        extra_start(t + 2, s)

                return carry

            lax.fori_loop(0, NCHUNK // 2, pair_body, 0)

        def no_extra(t, s):
            del t, s

        def zero_acc_a(g, carry):
            acc_a[pl.ds(g * LANES, LANES)] = jnp.zeros((LANES,), jnp.float32)
            return carry

        lax.fori_loop(0, HW // LANES, zero_acc_a, 0)

        def cnt_chunk(t, s):
            del t

            def grp(g, carry):
                for idx, wt in _decode_group(iwbuf, s, g):
                    plsc.addupdate_scatter(acc_a, [idx], wt)
                return carry

            lax.fori_loop(0, GROUPS, grp, 0)

        chunk_loop(cnt_chunk, no_extra, no_extra)

        def rloop(k, carry):
            ck = sid + k * NUM_SUBCORES

            @pl.when(ck < NRCHUNK)
            def _():
                off = pl.multiple_of(ck * RCHUNK, 8)

                def rgrp(g, carry2):
                    v = acc_a[pl.ds(off + g * LANES, LANES)]
                    rsbuf[pl.ds(g * LANES, LANES)] = (
                        1.0 / jnp.maximum(v, 0.001))
                    return carry2

                lax.fori_loop(0, RCHUNK // LANES, rgrp, 0)
                pltpu.sync_copy(rsbuf, sh_r.at[pl.ds(off, RCHUNK)])

            return carry

        lax.fori_loop(0, (NRCHUNK + NUM_SUBCORES - 1) // NUM_SUBCORES,
                      rloop, 0)

        plsc.subcore_barrier()

        c_base = sid * CPT

        for p in range(PASSES):
            c0 = c_base + 2 * p

            def x_src(t):
                off = pl.multiple_of(t * CHUNK, 8)
                return x_hbm.at[cid, pl.ds(c0, 2), pl.ds(off, CHUNK)]

            def start_x(t, s):
                pltpu.async_copy(x_src(t), xbuf.at[s], sem_x[s])

            def wait_x(t, s):
                pltpu.make_async_copy(x_src(t), xbuf.at[s], sem_x[s]).wait()

            def zero_accs(g, carry):
                z = jnp.zeros((LANES,), jnp.float32)
                acc_a[pl.ds(g * LANES, LANES)] = z
                acc_b[pl.ds(g * LANES, LANES)] = z
                return carry

            lax.fori_loop(0, HW // LANES, zero_accs, 0)

            def scat_chunk(t, s):
                del t

                def grp(g, carry):
                    base = g * LANES
                    x0 = xbuf[s, 0, pl.ds(base, LANES)]
                    x1 = xbuf[s, 1, pl.ds(base, LANES)]
                    for idx, wt in _decode_group(iwbuf, s, g):
                        plsc.addupdate_scatter(acc_a, [idx], wt * x0)
                        plsc.addupdate_scatter(acc_b, [idx], wt * x1)
                    return carry

                lax.fori_loop(0, GROUPS, grp, 0)

            chunk_loop(scat_chunk, start_x, wait_x)

            def o_dst(t):
                off = pl.multiple_of(t * CHUNK, 8)
                return out_hbm.at[cid, pl.ds(c0, 2), pl.ds(off, CHUNK)]

            def fin_body(tp, carry):
                for s in (0, 1):
                    t = tp * 2 + s
                    off = pl.multiple_of(t * CHUNK, 8)
                    pltpu.sync_copy(sh_r.at[pl.ds(off, CHUNK)], rbuf.at[s])

                    @pl.when(t >= 2)
                    def _():
                        pltpu.make_async_copy(obuf.at[s], o_dst(t - 2),
                                              sem_o[s]).wait()

                    def ogrp(g, carry2):
                        base = g * LANES
                        rv = rbuf[s, pl.ds(base, LANES)]
                        pbase = off + base
                        obuf[s, 0, pl.ds(base, LANES)] = (
                            acc_a[pl.ds(pbase, LANES)] * rv)
                        obuf[s, 1, pl.ds(base, LANES)] = (
                            acc_b[pl.ds(pbase, LANES)] * rv)
                        return carry2

                    lax.fori_loop(0, GROUPS, ogrp, 0)
                    pltpu.async_copy(obuf.at[s], o_dst(t), sem_o[s])
                return carry

            lax.fori_loop(0, NCHUNK // 2, fin_body, 0)
            for s in (0, 1):
                t = NCHUNK - 2 + s
                pltpu.make_async_copy(obuf.at[s], o_dst(t), sem_o[s]).wait()

    f = pl.kernel(
        body,
        out_type=jax.ShapeDtypeStruct((B, C, HW), jnp.float32),
        mesh=mesh,
        scratch_types=[
            pltpu.VMEM((HW,), jnp.float32),
            pltpu.VMEM((HW,), jnp.float32),
            pltpu.VMEM((2, 4, CHUNK), jnp.float32),
            pltpu.VMEM((2, 2, CHUNK), jnp.float32),
            pltpu.VMEM((2, CHUNK), jnp.float32),
            pltpu.VMEM((RCHUNK,), jnp.float32),
            pltpu.VMEM((2, 2, CHUNK), jnp.float32),
            pltpu.VMEM_SHARED((HW,), jnp.float32),
            pltpu.SemaphoreType.DMA,
            pltpu.SemaphoreType.DMA,
            pltpu.SemaphoreType.DMA,
            pltpu.SemaphoreType.DMA,
            pltpu.SemaphoreType.DMA,
            pltpu.SemaphoreType.DMA,
        ],
        compiler_params=pltpu.CompilerParams(needs_layout_passes=False),
    )
    return f(xr, iw)


@jax.jit
def kernel(x, phi):
    iw = _prep_pallas(phi).reshape(B, 4, HW)
    xr = x.reshape(B, C, HW)
    out = _sc_push(xr, iw)
    return out.reshape(B, C, H, W)


# speedup vs baseline: 5.4837x; 1.5020x over previous
import functools

import jax
import jax.numpy as jnp
from jax import lax
from jax.experimental import pallas as pl
from jax.experimental.pallas import tpu as pltpu
from jax.experimental.pallas import tpu_sc as plsc

B = 2
C = 192
H = 224
W = 224
HW = H * W

NUM_CORES = 2
NUM_SUBCORES = 16
LANES = 16

CPT = C // NUM_SUBCORES
PASSES = CPT // 2
CHUNK = 896
NCHUNK = HW // CHUNK
GROUPS = CHUNK // LANES


def _prep_pallas(phi):
    def body(phi_ref, i_ref, w_ref):
        ph0 = phi_ref[0]
        ph1 = phi_ref[1]
        ii = lax.broadcasted_iota(jnp.int32, (H, W), 0).astype(jnp.float32)
        jj = lax.broadcasted_iota(jnp.int32, (H, W), 1).astype(jnp.float32)
        gi = ii + ph0
        gj = jj + ph1
        i0f = jnp.floor(gi)
        j0f = jnp.floor(gj)
        i0 = jnp.mod(i0f.astype(jnp.int32), H)
        j0 = jnp.mod(j0f.astype(jnp.int32), W)
        i_ref[0] = i0 * W + j0
        i_ref[1] = jnp.where(j0 == W - 1, 1 - W, 1)
        i_ref[2] = jnp.where(i0 == H - 1, W - HW, W)
        w_ref[0] = gi - i0f
        w_ref[1] = gj - j0f

    return pl.pallas_call(
        body,
        grid=(B,),
        in_specs=[pl.BlockSpec((None, 2, H, W), lambda b: (b, 0, 0, 0))],
        out_specs=[
            pl.BlockSpec((None, 3, H, W), lambda b: (b, 0, 0, 0)),
            pl.BlockSpec((None, 2, H, W), lambda b: (b, 0, 0, 0)),
        ],
        out_shape=[
            jax.ShapeDtypeStruct((B, 3, H, W), jnp.int32),
            jax.ShapeDtypeStruct((B, 2, H, W), jnp.float32),
        ],
    )(phi)


def _norm_pallas(acc, cnt):
    CB = 16

    def body(a_ref, c_ref, o_ref):
        o_ref[...] = a_ref[...] / jnp.maximum(c_ref[...], 0.001)

    return pl.pallas_call(
        body,
        grid=(B, C // CB),
        in_specs=[
            pl.BlockSpec((None, CB, HW), lambda b, c: (b, c, 0)),
            pl.BlockSpec((None, 1, HW), lambda b, c: (b, 0, 0)),
        ],
        out_specs=pl.BlockSpec((None, CB, HW), lambda b, c: (b, c, 0)),
        out_shape=jax.ShapeDtypeStruct((B, C, HW), jnp.float32),
    )(acc, cnt)


def _decode_group(ibuf, wbuf, s, g):
    base = g * LANES
    idx00 = ibuf[s, 0, pl.ds(base, LANES)]
    dj = ibuf[s, 1, pl.ds(base, LANES)]
    di = ibuf[s, 2, pl.ds(base, LANES)]
    wi = wbuf[s, 0, pl.ds(base, LANES)]
    wj = wbuf[s, 1, pl.ds(base, LANES)]
    idx01 = idx00 + dj
    idx10 = idx00 + di
    idx11 = idx01 + di
    u = 1.0 - wi
    v = 1.0 - wj
    return (
        (idx00, u * v),
        (idx01, u * wj),
        (idx10, wi * v),
        (idx11, wi * wj),
    )


def _sc_push(xr, itab, wtab):
    mesh = plsc.VectorSubcoreMesh(core_axis_name="c", subcore_axis_name="s")

    def body(x_hbm, i_hbm, w_hbm, out_hbm, cnt_hbm, acc_a, acc_b, ibuf, wbuf,
             xbuf, sem_i0, sem_i1, sem_w0, sem_w1, sem_x0, sem_x1):
        cid = lax.axis_index("c")
        sid = lax.axis_index("s")
        sem_i = (sem_i0, sem_i1)
        sem_w = (sem_w0, sem_w1)
        sem_x = (sem_x0, sem_x1)

        def tab_src(t):
            return i_hbm.at[cid, t], w_hbm.at[cid, t]

        def start_tab(t, s):
            si, sw = tab_src(t)
            pltpu.async_copy(si, ibuf.at[s], sem_i[s])
            pltpu.async_copy(sw, wbuf.at[s], sem_w[s])

        def wait_tab(t, s):
            si, sw = tab_src(t)
            pltpu.make_async_copy(si, ibuf.at[s], sem_i[s]).wait()
            pltpu.make_async_copy(sw, wbuf.at[s], sem_w[s]).wait()

        def chunk_loop(compute_chunk, extra_start, extra_wait):
            start_tab(0, 0)
            extra_start(0, 0)
            start_tab(1, 1)
            extra_start(1, 1)

            def pair_body(tp, carry):
                for s in (0, 1):
                    t = tp * 2 + s
                    wait_tab(t, s)
                    extra_wait(t, s)
                    compute_chunk(t, s)

                    @pl.when(t + 2 < NCHUNK)
                    def _():
                        start_tab(t + 2, s)
                        extra_start(t + 2, s)

                return carry

            lax.fori_loop(0, NCHUNK // 2, pair_body, 0)

        def no_extra(t, s):
            del t, s

        def zero_a():
            @plsc.parallel_loop(0, HW // LANES, 1, unroll=8)
            def _(g):
                acc_a[pl.ds(g * LANES, LANES)] = jnp.zeros((LANES,),
                                                           jnp.float32)

        def zero_ab():
            @plsc.parallel_loop(0, HW // LANES, 1, unroll=8)
            def _(g):
                z = jnp.zeros((LANES,), jnp.float32)
                acc_a[pl.ds(g * LANES, LANES)] = z
                acc_b[pl.ds(g * LANES, LANES)] = z

        zero_a()

        def cnt_chunk(t, s):
            del t

            @plsc.parallel_loop(0, GROUPS, 1, unroll=4)
            def _(g):
                for idx, wt in _decode_group(ibuf, wbuf, s, g):
                    plsc.addupdate_scatter(acc_a, [idx], wt)

        chunk_loop(cnt_chunk, no_extra, no_extra)

        @pl.when(sid == 0)
        def _():
            pltpu.sync_copy(acc_a, cnt_hbm.at[cid])

        c_base = sid * CPT

        for p in range(PASSES):
            c0 = c_base + 2 * p

            def x_src(t):
                off = pl.multiple_of(t * CHUNK, 8)
                return x_hbm.at[cid, pl.ds(c0, 2), pl.ds(off, CHUNK)]

            def start_x(t, s):
                pltpu.async_copy(x_src(t), xbuf.at[s], sem_x[s])

            def wait_x(t, s):
                pltpu.make_async_copy(x_src(t), xbuf.at[s], sem_x[s]).wait()

            zero_ab()

            def scat_chunk(t, s):
                del t

                @plsc.parallel_loop(0, GROUPS, 1, unroll=4)
                def _(g):
                    base = g * LANES
                    x0 = xbuf[s, 0, pl.ds(base, LANES)]
                    x1 = xbuf[s, 1, pl.ds(base, LANES)]
                    for idx, wt in _decode_group(ibuf, wbuf, s, g):
                        plsc.addupdate_scatter(acc_a, [idx], wt * x0)
                        plsc.addupdate_scatter(acc_b, [idx], wt * x1)

            chunk_loop(scat_chunk, start_x, wait_x)

            pltpu.sync_copy(acc_a, out_hbm.at[cid, c0])
            pltpu.sync_copy(acc_b, out_hbm.at[cid, c0 + 1])

    f = pl.kernel(
        body,
        out_type=[
            jax.ShapeDtypeStruct((B, C, HW), jnp.float32),
            jax.ShapeDtypeStruct((B, HW), jnp.float32),
        ],
        mesh=mesh,
        scratch_types=[
            pltpu.VMEM((HW,), jnp.float32),
            pltpu.VMEM((HW,), jnp.float32),
            pltpu.VMEM((2, 3, CHUNK), jnp.int32),
            pltpu.VMEM((2, 2, CHUNK), jnp.float32),
            pltpu.VMEM((2, 2, CHUNK), jnp.float32),
            pltpu.SemaphoreType.DMA,
            pltpu.SemaphoreType.DMA,
            pltpu.SemaphoreType.DMA,
            pltpu.SemaphoreType.DMA,
            pltpu.SemaphoreType.DMA,
            pltpu.SemaphoreType.DMA,
        ],
        compiler_params=pltpu.CompilerParams(needs_layout_passes=False),
    )
    return f(xr, itab, wtab)


@jax.jit
def kernel(x, phi):
    itab, wtab = _prep_pallas(phi)
    xr = x.reshape(B, C, HW)
    itab = itab.reshape(B, 3, NCHUNK, CHUNK).transpose(0, 2, 1, 3)
    wtab = wtab.reshape(B, 2, NCHUNK, CHUNK).transpose(0, 2, 1, 3)
    acc, cnt = _sc_push(xr, itab, wtab)
    out = _norm_pallas(acc, cnt.reshape(B, 1, HW))
    return out.reshape(B, C, H, W)


# speedup vs baseline: 5.8299x; 1.0631x over previous
import functools

import jax
import jax.numpy as jnp
from jax import lax
from jax.experimental import pallas as pl
from jax.experimental.pallas import tpu as pltpu
from jax.experimental.pallas import tpu_sc as plsc

B = 2
C = 192
H = 224
W = 224
HW = H * W

NUM_CORES = 2
NUM_SUBCORES = 16
LANES = 16

CPT = C // NUM_SUBCORES
PASSES = CPT // 2
CHUNK = 896
NCHUNK = HW // CHUNK
GROUPS = CHUNK // LANES
NBUF = 4


def _prep_pallas(phi):
    def body(phi_ref, i_ref, w_ref):
        ph0 = phi_ref[0]
        ph1 = phi_ref[1]
        ii = lax.broadcasted_iota(jnp.int32, (H, W), 0).astype(jnp.float32)
        jj = lax.broadcasted_iota(jnp.int32, (H, W), 1).astype(jnp.float32)
        gi = ii + ph0
        gj = jj + ph1
        i0f = jnp.floor(gi)
        j0f = jnp.floor(gj)
        i0 = jnp.mod(i0f.astype(jnp.int32), H)
        j0 = jnp.mod(j0f.astype(jnp.int32), W)
        i_ref[0] = i0 * W + j0
        i_ref[1] = jnp.where(j0 == W - 1, 1 - W, 1)
        i_ref[2] = jnp.where(i0 == H - 1, W - HW, W)
        w_ref[0] = gi - i0f
        w_ref[1] = gj - j0f

    return pl.pallas_call(
        body,
        grid=(B,),
        in_specs=[pl.BlockSpec((None, 2, H, W), lambda b: (b, 0, 0, 0))],
        out_specs=[
            pl.BlockSpec((None, 3, H, W), lambda b: (b, 0, 0, 0)),
            pl.BlockSpec((None, 2, H, W), lambda b: (b, 0, 0, 0)),
        ],
        out_shape=[
            jax.ShapeDtypeStruct((B, 3, H, W), jnp.int32),
            jax.ShapeDtypeStruct((B, 2, H, W), jnp.float32),
        ],
    )(phi)


def _norm_pallas(acc, cnt):
    CB = 16

    def body(a_ref, c_ref, o_ref):
        total = jnp.sum(c_ref[...], axis=0, keepdims=True)
        o_ref[...] = a_ref[...] / jnp.maximum(total, 0.001)

    return pl.pallas_call(
        body,
        grid=(B, C // CB),
        in_specs=[
            pl.BlockSpec((None, CB, HW), lambda b, c: (b, c, 0)),
            pl.BlockSpec((None, NUM_SUBCORES, HW), lambda b, c: (b, 0, 0)),
        ],
        out_specs=pl.BlockSpec((None, CB, HW), lambda b, c: (b, c, 0)),
        out_shape=jax.ShapeDtypeStruct((B, C, HW), jnp.float32),
    )(acc, cnt)


def _decode_group(ibuf, wbuf, s, g):
    base = g * LANES
    idx00 = ibuf[s, 0, pl.ds(base, LANES)]
    dj = ibuf[s, 1, pl.ds(base, LANES)]
    di = ibuf[s, 2, pl.ds(base, LANES)]
    wi = wbuf[s, 0, pl.ds(base, LANES)]
    wj = wbuf[s, 1, pl.ds(base, LANES)]
    idx01 = idx00 + dj
    idx10 = idx00 + di
    idx11 = idx01 + di
    u = 1.0 - wi
    v = 1.0 - wj
    return (
        (idx00, u * v),
        (idx01, u * wj),
        (idx10, wi * v),
        (idx11, wi * wj),
    )


def _sc_push(xr, itab, wtab):
    mesh = plsc.VectorSubcoreMesh(core_axis_name="c", subcore_axis_name="s")

    def body(x_hbm, i_hbm, w_hbm, out_hbm, cnt_hbm, acc_a, acc_b, ibuf, wbuf,
             xbuf, *sems):
        cid = lax.axis_index("c")
        sid = lax.axis_index("s")
        sem_i = sems[0:NBUF]
        sem_w = sems[NBUF:2 * NBUF]
        sem_x = sems[2 * NBUF:3 * NBUF]

        def tab_src(t):
            return i_hbm.at[cid, t], w_hbm.at[cid, t]

        def start_tab(t, s):
            si, sw = tab_src(t)
            pltpu.async_copy(si, ibuf.at[s], sem_i[s])
            pltpu.async_copy(sw, wbuf.at[s], sem_w[s])

        def wait_tab(t, s):
            si, sw = tab_src(t)
            pltpu.make_async_copy(si, ibuf.at[s], sem_i[s]).wait()
            pltpu.make_async_copy(sw, wbuf.at[s], sem_w[s]).wait()

        def chunk_loop(compute_chunk, extra_start, extra_wait):
            for t0 in range(NBUF - 1):
                start_tab(t0, t0)
                extra_start(t0, t0)

            def quad_body(tq, carry):
                for s in range(NBUF):
                    t = tq * NBUF + s
                    sn = (s + NBUF - 1) % NBUF

                    @pl.when(t + NBUF - 1 < NCHUNK)
                    def _():
                        start_tab(t + NBUF - 1, sn)
                        extra_start(t + NBUF - 1, sn)

                    wait_tab(t, s)
                    extra_wait(t, s)
                    compute_chunk(t, s)

                return carry

            lax.fori_loop(0, NCHUNK // NBUF, quad_body, 0)

        def zero_a():
            @plsc.parallel_loop(0, HW // LANES, 1, unroll=8)
            def _(g):
                acc_a[pl.ds(g * LANES, LANES)] = jnp.zeros((LANES,),
                                                           jnp.float32)

        def zero_ab():
            @plsc.parallel_loop(0, HW // LANES, 1, unroll=8)
            def _(g):
                z = jnp.zeros((LANES,), jnp.float32)
                acc_a[pl.ds(g * LANES, LANES)] = z
                acc_b[pl.ds(g * LANES, LANES)] = z

        zero_a()

        lo = (sid * 7) >> 1
        hi = ((sid + 1) * 7) >> 1

        def cnt_body(t, carry):
            pltpu.sync_copy(i_hbm.at[cid, t], ibuf.at[0])
            pltpu.sync_copy(w_hbm.at[cid, t], wbuf.at[0])

            @plsc.parallel_loop(0, GROUPS, 1, unroll=4)
            def _(g):
                for idx, wt in _decode_group(ibuf, wbuf, 0, g):
                    plsc.addupdate_scatter(acc_a, [idx], wt)

            return carry

        lax.fori_loop(lo, hi, cnt_body, 0)
        pltpu.sync_copy(acc_a, cnt_hbm.at[cid, sid])

        c_base = sid * CPT

        for p in range(PASSES):
            c0 = c_base + 2 * p

            def x_src(t):
                off = pl.multiple_of(t * CHUNK, 8)
                return x_hbm.at[cid, pl.ds(c0, 2), pl.ds(off, CHUNK)]

            def start_x(t, s):
                pltpu.async_copy(x_src(t), xbuf.at[s], sem_x[s])

            def wait_x(t, s):
                pltpu.make_async_copy(x_src(t), xbuf.at[s], sem_x[s]).wait()

            zero_ab()

            def scat_chunk(t, s):
                del t

                @plsc.parallel_loop(0, GROUPS, 1, unroll=4)
                def _(g):
                    base = g * LANES
                    x0 = xbuf[s, 0, pl.ds(base, LANES)]
                    x1 = xbuf[s, 1, pl.ds(base, LANES)]
                    for idx, wt in _decode_group(ibuf, wbuf, s, g):
                        plsc.addupdate_scatter(acc_a, [idx], wt * x0)
                        plsc.addupdate_scatter(acc_b, [idx], wt * x1)

            chunk_loop(scat_chunk, start_x, wait_x)

            pltpu.sync_copy(acc_a, out_hbm.at[cid, c0])
            pltpu.sync_copy(acc_b, out_hbm.at[cid, c0 + 1])

    f = pl.kernel(
        body,
        out_type=[
            jax.ShapeDtypeStruct((B, C, HW), jnp.float32),
            jax.ShapeDtypeStruct((B, NUM_SUBCORES, HW), jnp.float32),
        ],
        mesh=mesh,
        scratch_types=[
            pltpu.VMEM((HW,), jnp.float32),
            pltpu.VMEM((HW,), jnp.float32),
            pltpu.VMEM((NBUF, 3, CHUNK), jnp.int32),
            pltpu.VMEM((NBUF, 2, CHUNK), jnp.float32),
            pltpu.VMEM((NBUF, 2, CHUNK), jnp.float32),
        ] + [pltpu.SemaphoreType.DMA] * (3 * NBUF),
        compiler_params=pltpu.CompilerParams(needs_layout_passes=False),
    )
    return f(xr, itab, wtab)


@jax.jit
def kernel(x, phi):
    itab, wtab = _prep_pallas(phi)
    xr = x.reshape(B, C, HW)
    itab = itab.reshape(B, 3, NCHUNK, CHUNK).transpose(0, 2, 1, 3)
    wtab = wtab.reshape(B, 2, NCHUNK, CHUNK).transpose(0, 2, 1, 3)
    acc, cnt = _sc_push(xr, itab, wtab)
    out = _norm_pallas(acc, cnt)
    return out.reshape(B, C, H, W)
